# baseline (speedup 1.0000x reference)
import jax
import jax.numpy as jnp
from jax import lax
from jax.experimental import pallas as pl
from jax.experimental.pallas import tpu as pltpu

NUM_BITS = 8
MIN_VAL = 0.0
MAX_VAL = 1.0
THRESHOLD = 3.0
B = 1024
F = 512
HIDDEN = 2048
CLASSES = 1000
ENC = F * NUM_BITS

BB = 128
HB = 256

NB = B // BB
NH = HIDDEN // HB
W0_DEPTH = 3
W1_DEPTH = 6
OC_DEPTH = 8
S_Z0 = NB
S_Z1 = S_Z0 + NH
S_OUT = S_Z1 + NH
S_ARGMAX = S_OUT + 2 * NH
N_STEPS = S_ARGMAX + 1


def _w0_copy(w0_ref, w0r, w0_sem, blk, slot):
    return pltpu.make_async_copy(
        w0_ref.at[pl.ds(blk * HB, HB), :], w0r.at[slot], w0_sem.at[slot])


def _w1_copy(w1_ref, w1s, w1_sem, blk, slot):
    return pltpu.make_async_copy(
        w1_ref.at[pl.ds(blk * HB, HB), :], w1s.at[slot], w1_sem.at[slot])


def _oc_copy(oc_ref, ocr, oc_sem, layer, hblk, slot):
    return pltpu.make_async_copy(
        oc_ref.at[layer, pl.ds(hblk * HB, HB), :], ocr.at[slot],
        oc_sem.at[slot])


def _body(x_ref, w0_ref, w1_ref, oc_ref, out_ref, pred_ref,
          enc_s, a0_s, a1_s, r_s, w0r, w1s, ocr, w0_sem, w1_sem, oc_sem):
    i = pl.program_id(0)

    @pl.when(i == 0)
    def _():
        for b in range(W0_DEPTH):
            _w0_copy(w0_ref, w0r, w0_sem, b, b).start()
        for b in range(W1_DEPTH):
            _w1_copy(w1_ref, w1s, w1_sem, b, b).start()
        for b in range(OC_DEPTH):
            _oc_copy(oc_ref, ocr, oc_sem, 0, b, b).start()
        src = lax.broadcasted_iota(jnp.int32, (F, ENC), 1) // NUM_BITS
        dst = lax.broadcasted_iota(jnp.int32, (F, ENC), 0)
        r_s[...] = (src == dst).astype(jnp.bfloat16)

    @pl.when(i < S_Z0)
    def _():
        xb = x_ref[...]
        xc = jnp.clip(xb, MIN_VAL, MAX_VAL)
        norm = (xc - MIN_VAL) / (MAX_VAL - MIN_VAL)
        lv = jnp.round(norm * (2 ** NUM_BITS - 1)).astype(jnp.int32)
        gray = lv ^ (lv >> 1)
        rep = lax.dot_general(gray.astype(jnp.bfloat16), r_s[...],
                              (((1,), (0,)), ((), ())),
                              preferred_element_type=jnp.float32)
        gi = rep.astype(jnp.int32)
        kidx = lax.broadcasted_iota(jnp.int32, (BB, ENC), 1) & (NUM_BITS - 1)
        enc_s[pl.ds(i * BB, BB), :] = ((gi >> kidx) & 1).astype(jnp.bfloat16)

    @pl.when((i >= S_Z0) & (i < S_Z1))
    def _():
        h = i - S_Z0
        slot = lax.rem(h, W0_DEPTH)
        _w0_copy(w0_ref, w0r, w0_sem, h, slot).wait()
        wb = w0r[slot].astype(jnp.bfloat16)
        z = lax.dot_general(enc_s[...], wb, (((1,), (1,)), ((), ())),
                            preferred_element_type=jnp.float32)
        a0_s[:, pl.ds(h * HB, HB)] = (z >= THRESHOLD).astype(jnp.bfloat16)

        @pl.when(h + W0_DEPTH < NH)
        def _():
            _w0_copy(w0_ref, w0r, w0_sem, h + W0_DEPTH, slot).start()

    @pl.when((i >= S_Z1) & (i < S_OUT))
    def _():
        h = i - S_Z1
        slot = lax.rem(h, W1_DEPTH)
        _w1_copy(w1_ref, w1s, w1_sem, h, slot).wait()
        wb = w1s[slot].astype(jnp.bfloat16)
        z = lax.dot_general(a0_s[...], wb, (((1,), (1,)), ((), ())),
                            preferred_element_type=jnp.float32)
        a1_s[:, pl.ds(h * HB, HB)] = (z >= THRESHOLD).astype(jnp.bfloat16)

        @pl.when(h + W1_DEPTH < NH)
        def _():
            _w1_copy(w1_ref, w1s, w1_sem, h + W1_DEPTH, slot).start()

    @pl.when((i >= S_OUT) & (i < S_ARGMAX))
    def _():
        j = i - S_OUT
        layer = j // NH
        h = lax.rem(j, NH)
        slot = lax.rem(j, OC_DEPTH)
        _oc_copy(oc_ref, ocr, oc_sem, layer, h, slot).wait()
        ocb = ocr[slot]

        def acc(a_s):
            ab = a_s[:, pl.ds(h * HB, HB)].astype(jnp.float32)
            return lax.dot_general(ab, ocb, (((1,), (0,)), ((), ())),
                                   preferred_element_type=jnp.float32)

        @pl.when(j == 0)
        def _():
            out_ref[...] = acc(a0_s)

        @pl.when((j > 0) & (j < NH))
        def _():
            out_ref[...] = out_ref[...] + acc(a0_s)

        @pl.when(j >= NH)
        def _():
            out_ref[...] = out_ref[...] + acc(a1_s)

        @pl.when(j + OC_DEPTH < 2 * NH)
        def _():
            _oc_copy(oc_ref, ocr, oc_sem, 1, h, slot).start()

    @pl.when(i == S_ARGMAX)
    def _():
        out = out_ref[...]
        mx = jnp.max(out, axis=1, keepdims=True)
        idx = lax.broadcasted_iota(jnp.int32, out.shape, 1)
        pred = jnp.min(jnp.where(out == mx, idx, CLASSES), axis=1)
        pred_ref[...] = pred.reshape(NB, 1, BB).astype(jnp.int32)


def kernel(trainOrTest, x, y, W0, W1, outC):
    del trainOrTest, y

    out_act, preds3 = pl.pallas_call(
        _body,
        grid=(N_STEPS,),
        in_specs=[
            pl.BlockSpec((BB, F), lambda i: (jnp.minimum(i, NB - 1), 0)),
            pl.BlockSpec(memory_space=pl.ANY),
            pl.BlockSpec(memory_space=pl.ANY),
            pl.BlockSpec(memory_space=pl.ANY),
        ],
        out_specs=[
            pl.BlockSpec((B, CLASSES), lambda i: (0, 0)),
            pl.BlockSpec((NB, 1, BB), lambda i: (0, 0, 0)),
        ],
        out_shape=[
            jax.ShapeDtypeStruct((B, CLASSES), jnp.float32),
            jax.ShapeDtypeStruct((NB, 1, BB), jnp.int32),
        ],
        scratch_shapes=[
            pltpu.VMEM((B, ENC), jnp.bfloat16),
            pltpu.VMEM((B, HIDDEN), jnp.bfloat16),
            pltpu.VMEM((B, HIDDEN), jnp.bfloat16),
            pltpu.VMEM((F, ENC), jnp.bfloat16),
            pltpu.VMEM((W0_DEPTH, HB, ENC), jnp.float32),
            pltpu.VMEM((W1_DEPTH, HB, HIDDEN), jnp.float32),
            pltpu.VMEM((OC_DEPTH, HB, CLASSES), jnp.float32),
            pltpu.SemaphoreType.DMA((W0_DEPTH,)),
            pltpu.SemaphoreType.DMA((W1_DEPTH,)),
            pltpu.SemaphoreType.DMA((OC_DEPTH,)),
        ],
    )(x, W0, W1, outC)

    predictions = preds3.reshape(B)
    return predictions, out_act
